# baseline (speedup 1.0000x reference)
import functools

import jax
import jax.numpy as jnp
from jax import lax
from jax.experimental import pallas as pl
from jax.experimental.pallas import tpu as pltpu
from jax.experimental.pallas import tpu_sc as plsc

N_NODES = 10000
N_EDGES = 320000
CH = 128

NC = 2
NS = 16
NW = NC * NS
K = 128
CHUNKS = 80
EPW = CHUNKS * K
E_PAD = NW * EPW
N_PAD = 10240
RPT = N_PAD // NS


def _sc_aggregate(x, row1, col1, zeros):
    mesh = plsc.VectorSubcoreMesh(core_axis_name="c", subcore_axis_name="s")

    @functools.partial(
        pl.kernel,
        mesh=mesh,
        out_type=jax.ShapeDtypeStruct((NC, N_PAD, CH), jnp.float32),
        scratch_types=[
            pltpu.VMEM((K,), jnp.int32),
            pltpu.VMEM((K,), jnp.int32),
            pltpu.VMEM((K,), jnp.int32),
            pltpu.VMEM((K,), jnp.int32),
            pltpu.VMEM((K, CH), jnp.float32),
            pltpu.VMEM((K, CH), jnp.float32),
            pltpu.VMEM_SHARED((N_PAD, CH), jnp.float32),
            pltpu.SemaphoreType.DMA,
            pltpu.SemaphoreType.DMA,
            pltpu.SemaphoreType.DMA,
            pltpu.SemaphoreType.DMA,
        ],
    )
    def agg_kernel(x_hbm, row_hbm, col_hbm, z_hbm, out_hbm,
                   cbuf0, cbuf1, rbuf0, rbuf1, gbuf0, gbuf1, acc,
                   sem_i0, sem_i1, sem_g0, sem_g1):
        c = lax.axis_index("c")
        s = lax.axis_index("s")
        wid = c * NS + s
        base = wid * EPW

        cbuf = (cbuf0, cbuf1)
        rbuf = (rbuf0, rbuf1)
        gbuf = (gbuf0, gbuf1)
        sem_i = (sem_i0, sem_i1)
        sem_g = (sem_g0, sem_g1)

        def issue_idx(k, b):
            off = base + k * K
            pltpu.async_copy(col_hbm.at[pl.ds(off, K)], cbuf[b], sem_i[b])
            pltpu.async_copy(row_hbm.at[pl.ds(off, K)], rbuf[b], sem_i[b])

        def wait_idx(k, b):
            off = base + k * K
            pltpu.make_async_copy(col_hbm.at[pl.ds(off, K)], cbuf[b],
                                  sem_i[b]).wait()
            pltpu.make_async_copy(row_hbm.at[pl.ds(off, K)], rbuf[b],
                                  sem_i[b]).wait()

        def issue_gather(b):
            pltpu.async_copy(x_hbm.at[cbuf[b]], gbuf[b], sem_g[b])

        def wait_gather(b):
            pltpu.make_async_copy(x_hbm.at[cbuf[b]], gbuf[b],
                                  sem_g[b]).wait()

        pltpu.sync_copy(z_hbm.at[pl.ds(s * RPT, RPT)],
                        acc.at[pl.ds(s * RPT, RPT)])
        plsc.subcore_barrier()

        def body(k, carry):
            issue_idx(k, 0)
            wait_idx(k, 0)
            issue_gather(0)
            wait_gather(0)
            pltpu.sync_copy(gbuf[0], acc.at[rbuf[0]], add=True)
            return carry

        lax.fori_loop(0, CHUNKS, body, 0)

        plsc.subcore_barrier()
        pltpu.sync_copy(acc.at[pl.ds(s * RPT, RPT)],
                        out_hbm.at[c, pl.ds(s * RPT, RPT)])

    return agg_kernel(x, row1, col1, zeros)


ROWS_BLK = 2000


def _mm_body(p_ref, w_ref, o_ref):
    acc = p_ref[0] + p_ref[1]
    o_ref[...] = lax.dot_general(
        acc, w_ref[...], (((1,), (1,)), ((), ())),
        preferred_element_type=jnp.float32)


def _tc_matmul(partials, W_fc):
    return pl.pallas_call(
        _mm_body,
        grid=(N_NODES // ROWS_BLK,),
        in_specs=[
            pl.BlockSpec((NC, ROWS_BLK, CH), lambda i: (0, i, 0)),
            pl.BlockSpec((CH, CH), lambda i: (0, 0)),
        ],
        out_specs=pl.BlockSpec((ROWS_BLK, CH), lambda i: (i, 0)),
        out_shape=jax.ShapeDtypeStruct((N_NODES, CH), jnp.float32),
    )(partials, W_fc)


def kernel(x, edge_index, edge_attr, W_fc, W_edge, W_att):
    del edge_attr, W_edge, W_att
    ei = edge_index.astype(jnp.int32)
    pad = E_PAD - N_EDGES
    row1 = jnp.concatenate([ei[0], jnp.full((pad,), N_NODES, jnp.int32)])
    col1 = jnp.concatenate([ei[1], jnp.zeros((pad,), jnp.int32)])
    zeros = jnp.zeros((N_PAD, CH), jnp.float32)
    partials = _sc_aggregate(x, row1, col1, zeros)
    return _tc_matmul(partials[:, :N_NODES, :], W_fc)


# speedup vs baseline: 3.0948x; 3.0948x over previous
import functools

import jax
import jax.numpy as jnp
from jax import lax
from jax.experimental import pallas as pl
from jax.experimental.pallas import tpu as pltpu
from jax.experimental.pallas import tpu_sc as plsc

N_NODES = 10000
N_EDGES = 320000
CH = 128

NC = 2
NS = 16
NW = NC * NS
K = 80
CHUNKS = 125
EPW = CHUNKS * K
E_PAD = NW * EPW
N_PAD = 10240
RPT = N_PAD // NS


def _sc_aggregate(x, row1, col1, zeros):
    mesh = plsc.VectorSubcoreMesh(core_axis_name="c", subcore_axis_name="s")

    @functools.partial(
        pl.kernel,
        mesh=mesh,
        out_type=jax.ShapeDtypeStruct((NC, N_PAD, CH), jnp.float32),
        scratch_types=[
            pltpu.VMEM((K,), jnp.int32),
            pltpu.VMEM((K,), jnp.int32),
            pltpu.VMEM((K,), jnp.int32),
            pltpu.VMEM((K,), jnp.int32),
            pltpu.VMEM((K, CH), jnp.float32),
            pltpu.VMEM((K, CH), jnp.float32),
            pltpu.VMEM_SHARED((N_PAD, CH), jnp.float32),
            pltpu.SemaphoreType.DMA,
            pltpu.SemaphoreType.DMA,
            pltpu.SemaphoreType.DMA,
            pltpu.SemaphoreType.DMA,
        ],
    )
    def agg_kernel(x_hbm, row_hbm, col_hbm, z_hbm, out_hbm,
                   cbuf0, cbuf1, rbuf0, rbuf1, gbuf0, gbuf1, acc,
                   sem_i0, sem_i1, sem_g0, sem_g1):
        c = lax.axis_index("c")
        s = lax.axis_index("s")
        wid = c * NS + s
        base = wid * EPW

        cbuf = (cbuf0, cbuf1)
        rbuf = (rbuf0, rbuf1)
        gbuf = (gbuf0, gbuf1)
        sem_i = (sem_i0, sem_i1)
        sem_g = (sem_g0, sem_g1)

        def issue_idx(k, b):
            off = base + k * K
            pltpu.async_copy(col_hbm.at[pl.ds(off, K)], cbuf[b], sem_i[b])
            pltpu.async_copy(row_hbm.at[pl.ds(off, K)], rbuf[b], sem_i[b])

        def wait_idx(k, b):
            off = base + k * K
            pltpu.make_async_copy(col_hbm.at[pl.ds(off, K)], cbuf[b],
                                  sem_i[b]).wait()
            pltpu.make_async_copy(row_hbm.at[pl.ds(off, K)], rbuf[b],
                                  sem_i[b]).wait()

        def issue_gather(b):
            pltpu.async_copy(x_hbm.at[cbuf[b]], gbuf[b], sem_g[b])

        def wait_gather(b):
            pltpu.make_async_copy(x_hbm.at[cbuf[b]], gbuf[b],
                                  sem_g[b]).wait()

        issue_idx(0, 0)
        pltpu.sync_copy(z_hbm.at[pl.ds(s * RPT, RPT)],
                        acc.at[pl.ds(s * RPT, RPT)])
        wait_idx(0, 0)
        issue_gather(0)
        plsc.subcore_barrier()

        def half(k, b):
            b2 = 1 - b
            issue_idx(k + 1, b2)
            wait_idx(k + 1, b2)
            issue_gather(b2)
            wait_gather(b)
            pltpu.sync_copy(gbuf[b], acc.at[rbuf[b]], add=True)

        def body(g, carry):
            half(g * 2, 0)
            half(g * 2 + 1, 1)
            return carry

        lax.fori_loop(0, (CHUNKS - 1) // 2, body, 0)
        wait_gather(0)
        pltpu.sync_copy(gbuf[0], acc.at[rbuf[0]], add=True)

        plsc.subcore_barrier()
        pltpu.sync_copy(acc.at[pl.ds(s * RPT, RPT)],
                        out_hbm.at[c, pl.ds(s * RPT, RPT)])

    return agg_kernel(x, row1, col1, zeros)


ROWS_BLK = 2000


def _mm_body(p_ref, w_ref, o_ref):
    acc = p_ref[0] + p_ref[1]
    o_ref[...] = lax.dot_general(
        acc, w_ref[...], (((1,), (1,)), ((), ())),
        preferred_element_type=jnp.float32)


def _tc_matmul(partials, W_fc):
    return pl.pallas_call(
        _mm_body,
        grid=(N_NODES // ROWS_BLK,),
        in_specs=[
            pl.BlockSpec((NC, ROWS_BLK, CH), lambda i: (0, i, 0)),
            pl.BlockSpec((CH, CH), lambda i: (0, 0)),
        ],
        out_specs=pl.BlockSpec((ROWS_BLK, CH), lambda i: (i, 0)),
        out_shape=jax.ShapeDtypeStruct((N_NODES, CH), jnp.float32),
    )(partials, W_fc)


def kernel(x, edge_index, edge_attr, W_fc, W_edge, W_att):
    del edge_attr, W_edge, W_att
    ei = edge_index.astype(jnp.int32)
    row1 = ei[0]
    col1 = ei[1]
    zeros = jnp.zeros((N_PAD, CH), jnp.float32)
    partials = _sc_aggregate(x, row1, col1, zeros)
    return _tc_matmul(partials[:, :N_NODES, :], W_fc)
